# baseline (speedup 1.0000x reference)
import functools
import math

import jax
import jax.numpy as jnp
from jax import lax
from jax.experimental import pallas as pl
from jax.experimental.pallas import tpu as pltpu
from jax.experimental.pallas import tpu_sc as plsc

VOCAB = 1000000
EMB = 64
B = 4096
L = 200

NC = 2
NS = 16
NW = NC * NS

TOT = B * L
PER_W = TOT // NW
CHUNK = 128
K = 4
GROUP = CHUNK * K
NG = PER_W // GROUP

SCALE = math.sqrt(EMB)


def _make_sc_kernel():
  mesh = plsc.VectorSubcoreMesh(core_axis_name="c", subcore_axis_name="s")

  @functools.partial(
      pl.kernel,
      mesh=mesh,
      out_type=jax.ShapeDtypeStruct((TOT, EMB), jnp.float32),
      compiler_params=pltpu.CompilerParams(use_tc_tiling_on_sc=False),
      scratch_types=[
          pltpu.VMEM((K, CHUNK), jnp.int32),
          pltpu.VMEM((GROUP, EMB), jnp.float32),
          pltpu.SemaphoreType.DMA,
      ],
  )
  def embed(idx_hbm, tab_hbm, out_hbm, idx_v, rows_v, gsem):
    wid = lax.axis_index("s") * NC + lax.axis_index("c")
    base = wid * PER_W

    def group_body(g, carry):
      pltpu.sync_copy(idx_hbm.at[wid, g], idx_v)
      for j in range(K):
        pltpu.async_copy(
            tab_hbm.at[idx_v.at[j]],
            rows_v.at[pl.ds(j * CHUNK, CHUNK)],
            gsem,
        )
      for j in range(K):
        pltpu.make_async_copy(
            tab_hbm.at[idx_v.at[j]],
            rows_v.at[pl.ds(j * CHUNK, CHUNK)],
            gsem,
        ).wait()

      def scale_row(i, c2):
        for c in range(EMB // 16):
          v = rows_v[i, pl.ds(c * 16, 16)]
          rows_v[i, pl.ds(c * 16, 16)] = v * SCALE
        return c2

      lax.fori_loop(0, GROUP, scale_row, 0)

      pltpu.sync_copy(rows_v, out_hbm.at[pl.ds(base + g * GROUP, GROUP)])
      return carry

    lax.fori_loop(0, NG, group_body, 0)

  return embed


_sc_embed = _make_sc_kernel()


@jax.jit
def kernel(tokens, table):
  idx = tokens.reshape(NW, NG, K, CHUNK).astype(jnp.int32)
  out = _sc_embed(idx, table)
  return out.reshape(B, L, EMB)
